# baseline (speedup 1.0000x reference)
import functools

import jax
import jax.numpy as jnp
from jax import lax
from jax.experimental import pallas as pl
from jax.experimental.pallas import tpu as pltpu
from jax.experimental.pallas import tpu_sc as plsc

_FEAT_DIMS = [119, 9, 11, 12, 9, 5, 8, 2, 2]
_HID_N = 512
_HID_E = 128
_N_NODES = 10000
_N_EDGES = 320000

_NW = 32
_NB_PAD = 10240
_CN = 16
_CE = 80

_NODE_OFFS = [0, 119, 128, 139, 151, 160, 165, 173, 175]
_EDGE_OFFS = [0, 119, 128]


def _body(xT, eT, Wn, We, node_out, edge_out,
          nidx, ntmp, eidx, ebufs, outstg, sem):
    info = plsc.get_sparse_core_info()
    nc = info.num_cores
    wid = lax.axis_index("s") * nc + lax.axis_index("c")

    nbase = wid * (_NB_PAD // _NW)

    def node_chunk(c, _):
        row0 = nbase + c * _CN
        for i in range(9):
            pltpu.sync_copy(xT.at[pl.ds(i * _NB_PAD + row0, _CN)], nidx.at[i])
            if _NODE_OFFS[i]:
                nidx[i, :] = nidx[i, :] + _NODE_OFFS[i]
        cps = [pltpu.async_copy(Wn.at[nidx.at[i]], ntmp.at[i], sem)
               for i in range(9)]
        for cp in cps:
            cp.wait()

        def sum_row(r, _):
            def sum_vec(k, _):
                s = ntmp[0, r, pl.ds(k * 16, 16)]
                for i in range(1, 9):
                    s = s + ntmp[i, r, pl.ds(k * 16, 16)]
                outstg[r, pl.ds(k * 16, 16)] = s
                return _
            return lax.fori_loop(0, _HID_N // 16, sum_vec, _)
        lax.fori_loop(0, _CN, sum_row, None)
        pltpu.sync_copy(outstg, node_out.at[pl.ds(row0, _CN)])
        return _
    lax.fori_loop(0, (_NB_PAD // _NW) // _CN, node_chunk, None)

    ebase = wid * (_N_EDGES // _NW)

    def edge_chunk(c, _):
        row0 = ebase + c * _CE
        for i in range(3):
            pltpu.sync_copy(eT.at[pl.ds(i * _N_EDGES + row0, _CE)], eidx.at[i])
            if _EDGE_OFFS[i]:
                def offs(v, _, i=i):
                    eidx[i, pl.ds(v * 16, 16)] = (
                        eidx[i, pl.ds(v * 16, 16)] + _EDGE_OFFS[i])
                    return _
                lax.fori_loop(0, _CE // 16, offs, _)
        cps = [pltpu.async_copy(We.at[eidx.at[i]], ebufs.at[i], sem)
               for i in range(3)]
        for cp in cps:
            cp.wait()

        def sum_row(r, _):
            def sum_vec(k, _):
                s = (ebufs[0, r, pl.ds(k * 16, 16)]
                     + ebufs[1, r, pl.ds(k * 16, 16)]
                     + ebufs[2, r, pl.ds(k * 16, 16)])
                ebufs[0, r, pl.ds(k * 16, 16)] = s
                return _
            return lax.fori_loop(0, _HID_E // 16, sum_vec, _)
        lax.fori_loop(0, _CE, sum_row, None)
        pltpu.sync_copy(ebufs.at[0], edge_out.at[pl.ds(row0, _CE)])
        return _
    lax.fori_loop(0, (_N_EDGES // _NW) // _CE, edge_chunk, None)


@jax.jit
def _run(xT, eT, Wn, We):
    mesh = plsc.VectorSubcoreMesh(core_axis_name="c", subcore_axis_name="s")
    kfn = pl.kernel(
        _body,
        out_type=(
            jax.ShapeDtypeStruct((_NB_PAD, _HID_N), jnp.float32),
            jax.ShapeDtypeStruct((_N_EDGES, _HID_E), jnp.float32),
        ),
        mesh=mesh,
        scratch_types=[
            pltpu.VMEM((9, _CN), jnp.int32),
            pltpu.VMEM((9, _CN, _HID_N), jnp.float32),
            pltpu.VMEM((3, _CE), jnp.int32),
            pltpu.VMEM((3, _CE, _HID_E), jnp.float32),
            pltpu.VMEM((_CN, _HID_N), jnp.float32),
            pltpu.SemaphoreType.DMA,
        ],
    )
    return kfn(xT, eT, Wn, We)


def kernel(x, edge_attr, W0, W1, W2, W3, W4, W5, W6, W7, W8, We0, We1, We2):
    Wn = jnp.concatenate([W0, W1, W2, W3, W4, W5, W6, W7, W8], axis=0)
    We = jnp.concatenate([We0, We1, We2], axis=0)
    xp = jnp.zeros((_NB_PAD, 9), jnp.int32).at[:_N_NODES].set(x)
    xT = xp.T.reshape(-1)
    eT = edge_attr.T.reshape(-1)
    node_out, edge_out = _run(xT, eT, Wn, We)
    return node_out[:_N_NODES], edge_out


# speedup vs baseline: 1.0369x; 1.0369x over previous
import jax
import jax.numpy as jnp
from jax import lax
from jax.experimental import pallas as pl
from jax.experimental.pallas import tpu as pltpu
from jax.experimental.pallas import tpu_sc as plsc

_HID_N = 512
_HID_E = 128
_N_NODES = 10000
_N_EDGES = 320000

_NW = 32
_NB_PAD = 10240
_CN = 8
_CE = 80
_NCN = (_NB_PAD // _NW) // _CN
_NCE = (_N_EDGES // _NW) // _CE


def _body(nIdxB, eI0, eI12, Wcat, Wecat, node_out, edge_out,
          nidx, nbuf, eidx, ebuf, sem0, sem1):
    info = plsc.get_sparse_core_info()
    nc = info.num_cores
    wid = lax.axis_index("s") * nc + lax.axis_index("c")
    sems = [sem0, sem1]

    ng0 = wid * _NCN

    def n_fire(cur, b):
        pltpu.sync_copy(nIdxB.at[pl.ds((ng0 + cur) * 40, 40)], nidx.at[b])
        pltpu.async_copy(Wcat.at[nidx.at[b]], nbuf.at[b], sems[b])

    def n_drain(b):
        pltpu.make_async_copy(Wcat.at[nidx.at[b]], nbuf.at[b], sems[b]).wait()

    def n_consume(cur, b):
        def sum_row(r, _):
            for k in range(_HID_N // 16):
                s = nbuf[b, r, pl.ds(k * 16, 16)]
                for j in range(1, 5):
                    s = s + nbuf[b, j * 8 + r, pl.ds(k * 16, 16)]
                nbuf[b, r, pl.ds(k * 16, 16)] = s
            return _
        lax.fori_loop(0, _CN, sum_row, None)
        row0 = (ng0 + cur) * _CN
        pltpu.sync_copy(nbuf.at[b, pl.ds(0, _CN)],
                        node_out.at[pl.ds(row0, _CN)])

    n_fire(0, 0)

    def n_step(i, _):
        for b in (0, 1):
            cur = i * 2 + b

            @pl.when(cur + 1 < _NCN)
            def _():
                n_fire(cur + 1, 1 - b)
            n_drain(b)
            n_consume(cur, b)
        return _
    lax.fori_loop(0, _NCN // 2, n_step, None)

    eg0 = wid * (_N_EDGES // _NW)

    def e_fire(cur, b):
        row0 = eg0 + cur * _CE
        pltpu.sync_copy(eI0.at[pl.ds(row0, _CE)], eidx.at[b, 0])
        pltpu.sync_copy(eI12.at[pl.ds(row0, _CE)], eidx.at[b, 1])
        pltpu.async_copy(Wecat.at[eidx.at[b, 0]], ebuf.at[b, 0], sems[b])
        pltpu.async_copy(Wecat.at[eidx.at[b, 1]], ebuf.at[b, 1], sems[b])

    def e_drain(b):
        for i in (0, 1):
            pltpu.make_async_copy(Wecat.at[eidx.at[b, i]], ebuf.at[b, i],
                                  sems[b]).wait()

    def e_consume(cur, b):
        def sum_row(r, _):
            for k in range(_HID_E // 16):
                s = (ebuf[b, 0, r, pl.ds(k * 16, 16)]
                     + ebuf[b, 1, r, pl.ds(k * 16, 16)])
                ebuf[b, 0, r, pl.ds(k * 16, 16)] = s
            return _
        lax.fori_loop(0, _CE, sum_row, None)
        row0 = eg0 + cur * _CE
        pltpu.sync_copy(ebuf.at[b, 0], edge_out.at[pl.ds(row0, _CE)])

    e_fire(0, 0)

    def e_step(i, _):
        for b in (0, 1):
            cur = i * 2 + b
            e_fire(cur + 1, 1 - b)
            e_drain(b)
            e_consume(cur, b)
        return _
    lax.fori_loop(0, (_NCE - 1) // 2, e_step, None)
    e_drain(0)
    e_consume(_NCE - 1, 0)


@jax.jit
def _run(nIdxB, eI0, eI12, Wcat, Wecat):
    mesh = plsc.VectorSubcoreMesh(core_axis_name="c", subcore_axis_name="s")
    kfn = pl.kernel(
        _body,
        out_type=(
            jax.ShapeDtypeStruct((_NB_PAD, _HID_N), jnp.float32),
            jax.ShapeDtypeStruct((_N_EDGES, _HID_E), jnp.float32),
        ),
        mesh=mesh,
        scratch_types=[
            pltpu.VMEM((2, 40), jnp.int32),
            pltpu.VMEM((2, 40, _HID_N), jnp.float32),
            pltpu.VMEM((2, 2, _CE), jnp.int32),
            pltpu.VMEM((2, 2, _CE, _HID_E), jnp.float32),
            pltpu.SemaphoreType.DMA,
            pltpu.SemaphoreType.DMA,
        ],
    )
    return kfn(nIdxB, eI0, eI12, Wcat, Wecat)


def kernel(x, edge_attr, W0, W1, W2, W3, W4, W5, W6, W7, W8, We0, We1, We2):
    Wc14 = (W1[:, None, :] + W4[None, :, :]).reshape(81, _HID_N)
    Wc23 = (W2[:, None, :] + W3[None, :, :]).reshape(132, _HID_N)
    Wc56 = (W5[:, None, :] + W6[None, :, :]).reshape(40, _HID_N)
    Wc78 = (W7[:, None, :] + W8[None, :, :]).reshape(4, _HID_N)
    Wcat = jnp.concatenate([W0, Wc14, Wc23, Wc56, Wc78], axis=0)
    Wec12 = (We1[:, None, :] + We2[None, :, :]).reshape(99, _HID_E)
    Wecat = jnp.concatenate([We0, Wec12], axis=0)

    xp = jnp.zeros((_NB_PAD, 9), jnp.int32).at[:_N_NODES].set(x)
    nI = jnp.stack([
        xp[:, 0],
        119 + xp[:, 1] * 9 + xp[:, 4],
        200 + xp[:, 2] * 12 + xp[:, 3],
        332 + xp[:, 5] * 8 + xp[:, 6],
        372 + xp[:, 7] * 2 + xp[:, 8],
    ], axis=1)
    nIdxB = nI.reshape(_NB_PAD // _CN, _CN, 5).transpose(0, 2, 1).reshape(-1)
    eI0 = edge_attr[:, 0]
    eI12 = 119 + edge_attr[:, 1] * 11 + edge_attr[:, 2]

    node_out, edge_out = _run(nIdxB, eI0, eI12, Wcat, Wecat)
    return node_out[:_N_NODES], edge_out
